# Initial kernel scaffold; baseline (speedup 1.0000x reference)
import jax
import jax.numpy as jnp
from jax.experimental import pallas as pl


def kernel(cards, suit_embedding, rank_embedding, card_embedding):
    raise NotImplementedError("write your pallas kernel here")


# speedup vs baseline: 235.5339x; 235.5339x over previous
import functools

import jax
import jax.numpy as jnp
from jax import lax
from jax.experimental import pallas as pl
from jax.experimental.pallas import tpu as pltpu
from jax.experimental.pallas import tpu_sc as plsc

_N_SUITS = 4
_N_RANKS = 13
_N_VALS = _N_SUITS * _N_RANKS
_D = 64
_N_CARDS = 819200

_NC = 2
_NS = 16
_NW = _NC * _NS
_L = 16

_CHUNK = _N_CARDS // _NW
_UNROLL = 8
_VECS = _CHUNK // _L


def _sc_body(cards_hbm, suit_hbm, rank_hbm, card_hbm, out_hbm,
             cards_v, suit_v, rank_v, card_v, partial_v):
    wid = lax.axis_index("s") * _NC + lax.axis_index("c")
    base = wid * _CHUNK

    pltpu.sync_copy(cards_hbm.at[pl.ds(base, _CHUNK)], cards_v)
    pltpu.sync_copy(suit_hbm, suit_v)
    pltpu.sync_copy(rank_hbm, rank_v)
    pltpu.sync_copy(card_hbm, card_v)

    def _hist(counts_v):
        lanes = lax.iota(jnp.int32, _L)
        ones = jnp.ones((_L,), jnp.float32)
        zeros = jnp.zeros((_L,), jnp.float32)
        for c in range(_N_VALS):
            counts_v[pl.ds(c * _L, _L)] = zeros

        def step(i, carry):
            start = i * (_L * _UNROLL)
            for k in range(_UNROLL):
                cv = cards_v[pl.ds(start + k * _L, _L)]
                plsc.addupdate_scatter(counts_v, [cv * _L + lanes], ones)
            return carry

        lax.fori_loop(0, _VECS // _UNROLL, step, 0)

        acc = [jnp.zeros((_L,), jnp.float32) for _ in range(_D // _L)]
        for c in range(_N_VALS):
            w = jnp.sum(counts_v[pl.ds(c * _L, _L)])
            for j in range(_D // _L):
                sl = pl.ds(j * _L, _L)
                t = suit_v[c // _N_RANKS, sl] + rank_v[c % _N_RANKS, sl] \
                    + card_v[c, sl]
                acc[j] = acc[j] + w * t
        for j in range(_D // _L):
            partial_v[pl.ds(j * _L, _L)] = acc[j]

    pl.run_scoped(_hist, pltpu.VMEM((_N_VALS * _L,), jnp.float32))
    pltpu.sync_copy(partial_v, out_hbm.at[wid])


@jax.jit
def kernel(cards, suit_embedding, rank_embedding, card_embedding):
    partials = pl.kernel(
        _sc_body,
        out_type=jax.ShapeDtypeStruct((_NW, _D), jnp.float32),
        mesh=plsc.VectorSubcoreMesh(core_axis_name="c", subcore_axis_name="s",
                                    num_cores=_NC, num_subcores=_NS),
        compiler_params=pltpu.CompilerParams(needs_layout_passes=False),
        scratch_types=[
            pltpu.VMEM((_CHUNK,), jnp.int32),
            pltpu.VMEM((_N_SUITS, _D), jnp.float32),
            pltpu.VMEM((_N_RANKS, _D), jnp.float32),
            pltpu.VMEM((_N_VALS, _D), jnp.float32),
            pltpu.VMEM((_D,), jnp.float32),
        ],
    )(cards, suit_embedding, rank_embedding, card_embedding)
    return jnp.sum(partials, axis=0)
